# baseline (speedup 1.0000x reference)
import functools

import jax
import jax.numpy as jnp
from jax import lax
from jax.experimental import pallas as pl
from jax.experimental.pallas import tpu as pltpu
from jax.experimental.pallas import tpu_sc as plsc

NUM_CLASSES = 1000000
OUT_FEATURES = 64
BATCH = 16384
PAIR = 2 * OUT_FEATURES
NUM_PAIRS = NUM_CLASSES // 2

_INFO = plsc.get_sparse_core_info()
_NC, _NS, _L = _INFO.num_cores, _INFO.num_subcores, _INFO.num_lanes
_NW = _NC * _NS
_BPW = BATCH // _NW
_CHUNK = 128
_NCHUNK = _BPW // _CHUNK

_mesh = plsc.VectorSubcoreMesh(core_axis_name="c", subcore_axis_name="s")


@functools.partial(
    pl.kernel,
    mesh=_mesh,
    out_type=jax.ShapeDtypeStruct((BATCH, OUT_FEATURES), jnp.float32),
    scratch_types=[
        pltpu.VMEM((_BPW,), jnp.int32),
        pltpu.VMEM((_BPW,), jnp.int32),
        pltpu.VMEM((2, _CHUNK, PAIR), jnp.float32),
        pltpu.VMEM((_BPW, OUT_FEATURES), jnp.float32),
        pltpu.SemaphoreType.DMA,
        pltpu.SemaphoreType.DMA,
    ],
    compiler_params=pltpu.CompilerParams(
        use_tc_tiling_on_sc=True, needs_layout_passes=False),
)
def _gather_kernel(idx_hbm, table_hbm, out_hbm,
                   idx_v, pid_v, pairs_v, out_v, sem0, sem1):
    wid = lax.axis_index("s") * _NC + lax.axis_index("c")
    base = wid * _BPW
    pltpu.sync_copy(idx_hbm.at[pl.ds(base, _BPW)], idx_v)

    def split(g, carry):
        off = g * _L
        pid_v[pl.ds(off, _L)] = lax.shift_right_logical(idx_v[pl.ds(off, _L)], 1)
        return carry

    lax.fori_loop(0, _BPW // _L, split, 0)

    copies = [None, None]
    sems = [sem0, sem1]

    def start(c):
        copies[c % 2] = pltpu.async_copy(
            table_hbm.at[pid_v.at[pl.ds(c * _CHUNK, _CHUNK)]],
            pairs_v.at[c % 2],
            sems[c % 2],
        )

    start(0)
    if _NCHUNK > 1:
        start(1)
    for c in range(_NCHUNK):
        copies[c % 2].wait()
        buf = pairs_v.at[c % 2]

        def select(g, carry, c=c, buf=buf):
            off = c * _CHUNK + g * _L
            jj = off + lax.iota(jnp.int32, _L)
            jl = g * _L + lax.iota(jnp.int32, _L)
            half = lax.bitwise_and(idx_v[pl.ds(off, _L)], 1) * OUT_FEATURES
            for col in range(OUT_FEATURES):
                cc = jnp.full((_L,), col, jnp.int32)
                vals = plsc.load_gather(buf, [jl, half + cc])
                plsc.store_scatter(out_v, [jj, cc], vals)
            return carry

        lax.fori_loop(0, _CHUNK // _L, select, 0)
        if c + 2 < _NCHUNK:
            start(c + 2)

    pltpu.sync_copy(out_v, out_hbm.at[pl.ds(base, _BPW)])


def kernel(class_ids, table):
    idx = class_ids.reshape(BATCH).astype(jnp.int32)
    table2 = table.reshape(NUM_PAIRS, PAIR)
    out = _gather_kernel(idx, table2)
    return out.reshape(BATCH, 1, OUT_FEATURES)
